# baseline (speedup 1.0000x reference)
import functools

import jax
import jax.numpy as jnp
from jax import lax
from jax.experimental import pallas as pl
from jax.experimental.pallas import tpu as pltpu
from jax.experimental.pallas import tpu_sc as plsc

NB = 4
BLK = 128


@functools.lru_cache(maxsize=None)
def _build(n_rows: int, seq: int, vocab: int, d: int):
    info = plsc.get_sparse_core_info()
    nc, ns, L = info.num_cores, info.num_subcores, info.num_lanes
    nw = nc * ns
    rows_per_w = n_rows // nw
    blocks_per_w = rows_per_w // BLK
    units_per_w = blocks_per_w * seq
    assert n_rows % (nw * BLK) == 0 and units_per_w % NB == 0
    assert d % 8 == 0 and BLK <= 128

    mesh = plsc.VectorSubcoreMesh(
        core_axis_name="c", subcore_axis_name="s",
        num_cores=nc, num_subcores=ns,
    )

    @functools.partial(
        pl.kernel,
        out_type=jax.ShapeDtypeStruct((seq, d // 8, n_rows // BLK, 8, BLK),
                                      jnp.float32),
        mesh=mesh,
        scratch_types=[
            pltpu.VMEM((rows_per_w, seq), jnp.int32),
            pltpu.VMEM((NB, BLK), jnp.int32),
            pltpu.VMEM((NB, BLK, d), jnp.float32),
            pltpu.VMEM((NB, d // 8, 8, BLK), jnp.float32),
        ] + [pltpu.SemaphoreType.DMA] * (2 * NB),
        compiler_params=pltpu.CompilerParams(use_tc_tiling_on_sc=False,
                                             needs_layout_passes=False),
    )
    def k(idx_hbm, table_hbm, out_hbm, idx_v, glist_v, grows_v, tbuf_v,
          *sems):
        gsem, ssem = sems[:NB], sems[NB:]
        wid = lax.axis_index("s") * nc + lax.axis_index("c")
        row0 = wid * rows_per_w
        blk0 = wid * blocks_per_w
        pltpu.sync_copy(idx_hbm.at[pl.ds(row0, rows_per_w)], idx_v)
        lanes = lax.iota(jnp.int32, L)

        def fill_glist(b, u):
            blk = u // seq
            j = u - blk * seq
            jv = jnp.full((L,), j, jnp.int32)

            @plsc.parallel_loop(0, BLK, step=L)
            def _(c0):
                iv = blk * BLK + c0 + lanes
                vals = plsc.load_gather(idx_v, [iv, jv])
                glist_v[b, pl.ds(c0, L)] = vals

        def issue_gather(b, u):
            return pltpu.async_copy(
                table_hbm.at[glist_v.at[b]], grows_v.at[b], gsem[b])

        trv = [(d0 + lanes) // 8 for d0 in range(0, d, L)]
        rv = [(d0 + lanes) % 8 for d0 in range(0, d, L)]

        def transpose(b):
            @plsc.parallel_loop(0, BLK, step=1, unroll=4)
            def _(c):
                cv = jnp.full((L,), c, jnp.int32)
                for k4 in range(d // L):
                    vals = grows_v[b, c, pl.ds(k4 * L, L)]
                    plsc.store_scatter(
                        tbuf_v.at[b], [trv[k4], rv[k4], cv], vals)

        def issue_store(b, u):
            blk = u // seq
            j = u - blk * seq
            return pltpu.async_copy(
                tbuf_v.at[b], out_hbm.at[j, :, blk0 + blk], ssem[b])

        for b in range(NB):
            fill_glist(b, b)
            issue_gather(b, b)

        def body(it, _):
            u0 = it * NB
            for b in range(NB):
                u = u0 + b
                pltpu.make_async_copy(
                    table_hbm.at[glist_v.at[b]], grows_v.at[b],
                    gsem[b]).wait()

                @pl.when(it > 0)
                def _():
                    pltpu.make_async_copy(
                        tbuf_v.at[b], out_hbm.at[0, :, 0], ssem[b]).wait()

                transpose(b)
                issue_store(b, u)

                @pl.when(u + NB < units_per_w)
                def _():
                    fill_glist(b, u + NB)
                    issue_gather(b, u + NB)
            return 0

        lax.fori_loop(0, units_per_w // NB, body, 0)
        for b in range(NB):
            pltpu.make_async_copy(
                tbuf_v.at[b], out_hbm.at[0, :, 0], ssem[b]).wait()

    return k


def kernel(token_ids, weight):
    n_rows, seq = token_ids.shape
    d = weight.shape[1]
    k = _build(n_rows, seq, weight.shape[0], d)
    out5d = k(token_ids.astype(jnp.int32), weight)
    return out5d.transpose(2, 4, 0, 1, 3).reshape(n_rows, seq, d)


# speedup vs baseline: 1.6248x; 1.6248x over previous
import functools

import jax
import jax.numpy as jnp
from jax import lax
from jax.experimental import pallas as pl
from jax.experimental.pallas import tpu as pltpu
from jax.experimental.pallas import tpu_sc as plsc

NB = 4
BLK = 128


@functools.lru_cache(maxsize=None)
def _build(n_rows: int, seq: int, vocab: int, d: int):
    info = plsc.get_sparse_core_info()
    nc, ns, L = info.num_cores, info.num_subcores, info.num_lanes
    nw = nc * ns
    rows_per_w = n_rows // nw
    blocks_per_w = rows_per_w // BLK
    units_per_w = blocks_per_w * seq
    assert n_rows % (nw * BLK) == 0 and units_per_w % NB == 0
    assert d % 8 == 0 and BLK <= 128

    mesh = plsc.VectorSubcoreMesh(
        core_axis_name="c", subcore_axis_name="s",
        num_cores=nc, num_subcores=ns,
    )

    @functools.partial(
        pl.kernel,
        out_type=jax.ShapeDtypeStruct((seq, d // 8, n_rows // BLK, 8, BLK),
                                      jnp.float32),
        mesh=mesh,
        scratch_types=[
            pltpu.VMEM((rows_per_w, seq), jnp.int32),
            pltpu.VMEM((NB, BLK), jnp.int32),
            pltpu.VMEM((NB, BLK, d), jnp.float32),
            pltpu.VMEM((NB, d // 8, 8, BLK), jnp.float32),
        ] + [pltpu.SemaphoreType.DMA] * (2 * NB),
        compiler_params=pltpu.CompilerParams(use_tc_tiling_on_sc=False,
                                             needs_layout_passes=False),
    )
    def k(idx_hbm, table_hbm, out_hbm, idx_v, glist_v, grows_v, tbuf_v,
          *sems):
        gsem, ssem = sems[:NB], sems[NB:]
        wid = lax.axis_index("s") * nc + lax.axis_index("c")
        row0 = wid * rows_per_w
        blk0 = wid * blocks_per_w
        pltpu.sync_copy(idx_hbm.at[pl.ds(row0, rows_per_w)], idx_v)
        lanes = lax.iota(jnp.int32, L)

        def fill_glist(b, u):
            blk = u // seq
            j = u - blk * seq
            jv = jnp.full((L,), j, jnp.int32)

            @plsc.parallel_loop(0, BLK, step=L)
            def _(c0):
                iv = blk * BLK + c0 + lanes
                vals = plsc.load_gather(idx_v, [iv, jv])
                glist_v[b, pl.ds(c0, L)] = vals

        def issue_gather(b, u):
            return pltpu.async_copy(
                table_hbm.at[glist_v.at[b]], grows_v.at[b], gsem[b])

        cvs = [c0 + lanes for c0 in range(0, BLK, L)]

        def transpose(b):
            @plsc.parallel_loop(0, L, step=1, unroll=4)
            def _(s):
                perm = (lanes + s) & (L - 1)
                for d0 in range(0, d, L):
                    rowoff = d0 + perm
                    trv = rowoff // 8
                    rvv = rowoff % 8
                    for ci in range(BLK // L):
                        vals = plsc.load_gather(grows_v.at[b],
                                                [cvs[ci], rowoff])
                        plsc.store_scatter(tbuf_v.at[b],
                                           [trv, rvv, cvs[ci]], vals)

        def issue_store(b, u):
            blk = u // seq
            j = u - blk * seq
            return pltpu.async_copy(
                tbuf_v.at[b], out_hbm.at[j, :, blk0 + blk], ssem[b])

        for b in range(NB):
            fill_glist(b, b)
            issue_gather(b, b)

        def body(it, _):
            u0 = it * NB
            for b in range(NB):
                u = u0 + b
                pltpu.make_async_copy(
                    table_hbm.at[glist_v.at[b]], grows_v.at[b],
                    gsem[b]).wait()

                @pl.when(it > 0)
                def _():
                    pltpu.make_async_copy(
                        tbuf_v.at[b], out_hbm.at[0, :, 0], ssem[b]).wait()

                transpose(b)
                issue_store(b, u)

                @pl.when(u + NB < units_per_w)
                def _():
                    fill_glist(b, u + NB)
                    issue_gather(b, u + NB)
            return 0

        lax.fori_loop(0, units_per_w // NB, body, 0)
        for b in range(NB):
            pltpu.make_async_copy(
                tbuf_v.at[b], out_hbm.at[0, :, 0], ssem[b]).wait()

    return k


def kernel(token_ids, weight):
    n_rows, seq = token_ids.shape
    d = weight.shape[1]
    k = _build(n_rows, seq, weight.shape[0], d)
    out5d = k(token_ids.astype(jnp.int32), weight)
    return out5d.transpose(2, 4, 0, 1, 3).reshape(n_rows, seq, d)
